# Initial kernel scaffold; baseline (speedup 1.0000x reference)
import functools

import jax
import jax.numpy as jnp
from jax import lax
from jax.experimental import pallas as pl
from jax.experimental.pallas import tpu as pltpu
from jax.experimental.pallas import tpu_sc as plsc

NC = 2
NS = 16
NW = NC * NS
K = 128


def _sc_gat_pass(hs, asrc, adst, src3, dst3, n_pad, e_real):
    n_nodes, feat = hs.shape
    cpt = src3.shape[1]
    ept = cpt * K
    rows_per_tile = n_pad // NS
    blocks_per_tile = rows_per_tile // K

    mesh = plsc.VectorSubcoreMesh(core_axis_name="c", subcore_axis_name="s")

    @functools.partial(
        pl.kernel,
        out_type=(
            jax.ShapeDtypeStruct((NC, n_pad, feat), jnp.float32),
            jax.ShapeDtypeStruct((NC, n_pad), jnp.float32),
        ),
        mesh=mesh,
        scratch_types=dict(
            src_v=pltpu.VMEM((cpt, K), jnp.int32),
            dst_v=pltpu.VMEM((cpt, K), jnp.int32),
            ea_v=pltpu.VMEM((ept,), jnp.float32),
            as_v=pltpu.VMEM((n_nodes,), jnp.float32),
            ad_v=pltpu.VMEM((n_nodes,), jnp.float32),
            rows_v=pltpu.VMEM((K, feat), jnp.float32),
            zden_v=pltpu.VMEM((rows_per_tile,), jnp.float32),
            out_acc=pltpu.VMEM_SHARED((n_pad, feat), jnp.float32),
            den_acc=pltpu.VMEM_SHARED((n_pad,), jnp.float32),
            sem=pltpu.SemaphoreType.DMA,
        ),
    )
    def gat_kernel(hs_hbm, as_hbm, ad_hbm, src_hbm, dst_hbm, outp_hbm, denp_hbm,
                   src_v, dst_v, ea_v, as_v, ad_v, rows_v, zden_v,
                   out_acc, den_acc, sem):
        cidx = lax.axis_index("c")
        sidx = lax.axis_index("s")
        wid = cidx * NS + sidx
        base_gid = wid * ept

        pltpu.sync_copy(src_hbm.at[wid], src_v)
        pltpu.sync_copy(dst_hbm.at[wid], dst_v)
        pltpu.sync_copy(as_hbm, as_v)
        pltpu.sync_copy(ad_hbm, ad_v)

        zero16 = jnp.zeros((16,), jnp.float32)

        def zrow(r, _):
            for v in range(feat // 16):
                rows_v[r, pl.ds(v * 16, 16)] = zero16
            return 0
        lax.fori_loop(0, K, zrow, 0)

        def zden(i, _):
            zden_v[pl.ds(i * 16, 16)] = zero16
            return 0
        lax.fori_loop(0, rows_per_tile // 16, zden, 0)

        for b in range(blocks_per_tile):
            pltpu.sync_copy(rows_v, out_acc.at[pl.ds(sidx * rows_per_tile + b * K, K)])
        pltpu.sync_copy(zden_v, den_acc.at[pl.ds(sidx * rows_per_tile, rows_per_tile)])
        plsc.subcore_barrier()

        iota16 = lax.iota(jnp.int32, 16)

        def chunk(j, _):
            pltpu.async_copy(hs_hbm.at[src_v.at[j]], rows_v, sem).wait()

            for g in range(K // 16):
                s16 = src_v[j, pl.ds(g * 16, 16)]
                d16 = dst_v[j, pl.ds(g * 16, 16)]
                a = plsc.load_gather(as_v, [s16]) + plsc.load_gather(ad_v, [d16])
                a = jnp.where(a >= 0.0, a, a * jnp.float32(0.2))
                ea = jnp.exp(a)
                gid = base_gid + j * K + g * 16 + iota16
                ea = jnp.where(gid < e_real, ea, jnp.float32(0.0))
                ea_v[pl.ds(j * K + g * 16, 16)] = ea

            def edge(e, _):
                sp = plsc.load_gather(ea_v, [jnp.full((16,), j * K, jnp.int32) + e])
                for v in range(feat // 16):
                    rows_v[e, pl.ds(v * 16, 16)] = rows_v[e, pl.ds(v * 16, 16)] * sp
                return 0
            lax.fori_loop(0, K, edge, 0)

            pltpu.sync_copy(ea_v.at[pl.ds(j * K, K)], den_acc.at[dst_v.at[j]], add=True)
            pltpu.sync_copy(rows_v, out_acc.at[dst_v.at[j]], add=True)
            return 0

        lax.fori_loop(0, cpt, chunk, 0)
        plsc.subcore_barrier()

        pltpu.sync_copy(out_acc.at[pl.ds(sidx * rows_per_tile, rows_per_tile)],
                        outp_hbm.at[cidx, pl.ds(sidx * rows_per_tile, rows_per_tile)])
        pltpu.sync_copy(den_acc.at[pl.ds(sidx * rows_per_tile, rows_per_tile)],
                        denp_hbm.at[cidx, pl.ds(sidx * rows_per_tile, rows_per_tile)])

    return gat_kernel(hs, asrc, adst, src3, dst3)


def _tc1_body(x_ref, we_ref, ws_ref, wd_ref, wl_ref, atts_ref, attd_ref,
              hs_ref, as_ref, ad_ref, hl_ref):
    h = jnp.dot(x_ref[...], we_ref[...], preferred_element_type=jnp.float32)
    hs = jnp.dot(h, ws_ref[...], preferred_element_type=jnp.float32)
    hs_ref[...] = hs
    as_ref[...] = jnp.dot(hs, atts_ref[...], preferred_element_type=jnp.float32)
    vd = jnp.dot(wd_ref[...], attd_ref[...], preferred_element_type=jnp.float32)
    ad_ref[...] = jnp.dot(h, vd, preferred_element_type=jnp.float32)
    hl_ref[...] = jnp.dot(h, wl_ref[...], preferred_element_type=jnp.float32)


def _tc2_body(outp_ref, denp_ref, b0_ref, hl0_ref, w1_ref, atts_ref, attd_ref,
              wlin1_ref, hs1_ref, as1_ref, ad1_ref, hl1_ref, *, n_nodes):
    num = outp_ref[0, :n_nodes, :] + outp_ref[1, :n_nodes, :]
    den = denp_ref[0, :n_nodes] + denp_ref[1, :n_nodes]
    c0 = num / (den + jnp.float32(1e-16))[:, None] + b0_ref[...][None, :]
    h1 = jax.nn.relu(c0 + hl0_ref[...])
    hs1 = jnp.dot(h1, w1_ref[...], preferred_element_type=jnp.float32)
    hs1_ref[...] = hs1
    as1_ref[...] = jnp.dot(hs1, atts_ref[...], preferred_element_type=jnp.float32)
    ad1_ref[...] = jnp.dot(hs1, attd_ref[...], preferred_element_type=jnp.float32)
    hl1_ref[...] = jnp.dot(h1, wlin1_ref[...], preferred_element_type=jnp.float32)


def _tc3_body(outp_ref, denp_ref, b1_ref, hl1_ref, lnw_ref, lnb_ref, wp_ref,
              out_ref, *, n_nodes):
    num = outp_ref[0, :n_nodes, :] + outp_ref[1, :n_nodes, :]
    den = denp_ref[0, :n_nodes] + denp_ref[1, :n_nodes]
    c1 = num / (den + jnp.float32(1e-16))[:, None] + b1_ref[...][None, :]
    h = c1 + hl1_ref[...]
    mu = jnp.mean(h)
    var = jnp.mean((h - mu) ** 2)
    h = (h - mu) / jnp.sqrt(var + jnp.float32(1e-5))
    h = h * lnw_ref[...][None, :] + lnb_ref[...][None, :]
    out_ref[...] = jnp.dot(h, wp_ref[...], preferred_element_type=jnp.float32)


def kernel(x, edge_index, W_embed, W_src0, W_dst0, att_src0, att_dst0, b0,
           W_lin0, W1, att_src1, att_dst1, b1, W_lin1, ln_w, ln_b, W_proj):
    n_nodes = x.shape[0]
    e_real = edge_index.shape[1]
    c0 = W_src0.shape[1]
    c1 = W1.shape[1]
    d_out = W_proj.shape[1]

    cpt = -(-e_real // (NW * K))
    e_pad = NW * cpt * K
    pad = e_pad - e_real
    src = edge_index[0]
    dst = edge_index[1]
    if pad:
        src = jnp.concatenate([src, jnp.arange(pad, dtype=jnp.int32) % n_nodes])
        dst = jnp.concatenate([dst, jnp.zeros((pad,), jnp.int32)])
    src3 = src.reshape(NW, cpt, K)
    dst3 = dst.reshape(NW, cpt, K)

    n_pad = NS * K * (-(-n_nodes // (NS * K)))

    hs0, as0, ad0, hl0 = pl.pallas_call(
        _tc1_body,
        out_shape=(
            jax.ShapeDtypeStruct((n_nodes, c0), jnp.float32),
            jax.ShapeDtypeStruct((n_nodes,), jnp.float32),
            jax.ShapeDtypeStruct((n_nodes,), jnp.float32),
            jax.ShapeDtypeStruct((n_nodes, c0), jnp.float32),
        ),
    )(x, W_embed, W_src0, W_dst0, W_lin0, att_src0, att_dst0)

    outp0, denp0 = _sc_gat_pass(hs0, as0, ad0, src3, dst3, n_pad, e_real)

    hs1, as1, ad1, hl1 = pl.pallas_call(
        functools.partial(_tc2_body, n_nodes=n_nodes),
        out_shape=(
            jax.ShapeDtypeStruct((n_nodes, c1), jnp.float32),
            jax.ShapeDtypeStruct((n_nodes,), jnp.float32),
            jax.ShapeDtypeStruct((n_nodes,), jnp.float32),
            jax.ShapeDtypeStruct((n_nodes, c1), jnp.float32),
        ),
    )(outp0, denp0, b0, hl0, W1, att_src1, att_dst1, W_lin1)

    outp1, denp1 = _sc_gat_pass(hs1, as1, ad1, src3, dst3, n_pad, e_real)

    out = pl.pallas_call(
        functools.partial(_tc3_body, n_nodes=n_nodes),
        out_shape=jax.ShapeDtypeStruct((n_nodes, d_out), jnp.float32),
    )(outp1, denp1, b1, hl1, ln_w, ln_b, W_proj)
    return out


# speedup vs baseline: 32.0648x; 32.0648x over previous
import functools

import jax
import jax.numpy as jnp
from jax import lax
from jax.experimental import pallas as pl
from jax.experimental.pallas import tpu as pltpu
from jax.experimental.pallas import tpu_sc as plsc

NC = 2
NS = 16
NW = NC * NS
K = 128


def _sc_gat_pass(hs, asrc, adst, src3, dst3, n_pad, e_real):
    n_nodes, feat = hs.shape
    cpt = src3.shape[1]
    ept = cpt * K
    rows_per_tile = n_pad // NS
    blocks_per_tile = rows_per_tile // K

    mesh = plsc.VectorSubcoreMesh(core_axis_name="c", subcore_axis_name="s")

    @functools.partial(
        pl.kernel,
        out_type=(
            jax.ShapeDtypeStruct((NC, n_pad, feat), jnp.float32),
            jax.ShapeDtypeStruct((NC, n_pad), jnp.float32),
        ),
        mesh=mesh,
        compiler_params=pltpu.CompilerParams(
            needs_layout_passes=False, use_tc_tiling_on_sc=False),
        scratch_types=dict(
            src_v=pltpu.VMEM((cpt, K), jnp.int32),
            dst_v=pltpu.VMEM((cpt, K), jnp.int32),
            ea_c=pltpu.VMEM((K,), jnp.float32),
            asg_v=pltpu.VMEM((K,), jnp.float32),
            adg_v=pltpu.VMEM((K,), jnp.float32),
            rows_v=pltpu.VMEM((K, feat), jnp.float32),
            zden_v=pltpu.VMEM((rows_per_tile,), jnp.float32),
            as_sh=pltpu.VMEM_SHARED((n_nodes,), jnp.float32),
            ad_sh=pltpu.VMEM_SHARED((n_nodes,), jnp.float32),
            out_acc=pltpu.VMEM_SHARED((n_pad, feat), jnp.float32),
            den_acc=pltpu.VMEM_SHARED((n_pad,), jnp.float32),
            sem=pltpu.SemaphoreType.DMA,
        ),
    )
    def gat_kernel(hs_hbm, as_hbm, ad_hbm, src_hbm, dst_hbm, outp_hbm, denp_hbm,
                   src_v, dst_v, ea_c, asg_v, adg_v, rows_v, zden_v,
                   as_sh, ad_sh, out_acc, den_acc, sem):
        cidx = lax.axis_index("c")
        sidx = lax.axis_index("s")
        wid = cidx * NS + sidx
        base_gid = wid * ept

        pltpu.sync_copy(src_hbm.at[wid], src_v)
        pltpu.sync_copy(dst_hbm.at[wid], dst_v)

        @pl.when(sidx == 0)
        def _():
            pltpu.sync_copy(as_hbm, as_sh)
            pltpu.sync_copy(ad_hbm, ad_sh)

        zero16 = jnp.zeros((16,), jnp.float32)

        def zrow(r, _):
            for v in range(feat // 16):
                rows_v[r, pl.ds(v * 16, 16)] = zero16
            return 0
        lax.fori_loop(0, K, zrow, 0)

        def zden(i, _):
            zden_v[pl.ds(i * 16, 16)] = zero16
            return 0
        lax.fori_loop(0, rows_per_tile // 16, zden, 0)

        for b in range(blocks_per_tile):
            pltpu.sync_copy(rows_v, out_acc.at[pl.ds(sidx * rows_per_tile + b * K, K)])
        pltpu.sync_copy(zden_v, den_acc.at[pl.ds(sidx * rows_per_tile, rows_per_tile)])
        plsc.subcore_barrier()

        iota16 = lax.iota(jnp.int32, 16)

        def chunk(j, _):
            rows_cp = pltpu.async_copy(hs_hbm.at[src_v.at[j]], rows_v, sem)
            pltpu.sync_copy(as_sh.at[src_v.at[j]], asg_v)
            pltpu.sync_copy(ad_sh.at[dst_v.at[j]], adg_v)

            for g in range(K // 16):
                a = asg_v[pl.ds(g * 16, 16)] + adg_v[pl.ds(g * 16, 16)]
                a = jnp.where(a >= 0.0, a, a * jnp.float32(0.2))
                ea = jnp.exp(a)
                gid = base_gid + j * K + g * 16 + iota16
                ea = jnp.where(gid < e_real, ea, jnp.float32(0.0))
                ea_c[pl.ds(g * 16, 16)] = ea

            rows_cp.wait()

            def edge(e, _):
                sp = plsc.load_gather(ea_c, [jnp.full((16,), e, jnp.int32)])
                for v in range(feat // 16):
                    rows_v[e, pl.ds(v * 16, 16)] = rows_v[e, pl.ds(v * 16, 16)] * sp
                return 0
            lax.fori_loop(0, K, edge, 0)

            pltpu.sync_copy(ea_c, den_acc.at[dst_v.at[j]], add=True)
            pltpu.sync_copy(rows_v, out_acc.at[dst_v.at[j]], add=True)
            return 0

        lax.fori_loop(0, cpt, chunk, 0)
        plsc.subcore_barrier()

        pltpu.sync_copy(out_acc.at[pl.ds(sidx * rows_per_tile, rows_per_tile)],
                        outp_hbm.at[cidx, pl.ds(sidx * rows_per_tile, rows_per_tile)])
        pltpu.sync_copy(den_acc.at[pl.ds(sidx * rows_per_tile, rows_per_tile)],
                        denp_hbm.at[cidx, pl.ds(sidx * rows_per_tile, rows_per_tile)])

    return gat_kernel(hs, asrc, adst, src3, dst3)


def _tc1_body(x_ref, we_ref, ws_ref, wd_ref, wl_ref, atts_ref, attd_ref,
              hs_ref, as_ref, ad_ref, hl_ref):
    h = jnp.dot(x_ref[...], we_ref[...], preferred_element_type=jnp.float32)
    hs = jnp.dot(h, ws_ref[...], preferred_element_type=jnp.float32)
    hs_ref[...] = hs
    as_ref[...] = jnp.dot(hs, atts_ref[...], preferred_element_type=jnp.float32)
    vd = jnp.dot(wd_ref[...], attd_ref[...], preferred_element_type=jnp.float32)
    ad_ref[...] = jnp.dot(h, vd, preferred_element_type=jnp.float32)
    hl_ref[...] = jnp.dot(h, wl_ref[...], preferred_element_type=jnp.float32)


def _tc2_body(outp_ref, denp_ref, b0_ref, hl0_ref, w1_ref, atts_ref, attd_ref,
              wlin1_ref, hs1_ref, as1_ref, ad1_ref, hl1_ref, *, n_nodes):
    num = outp_ref[0, :n_nodes, :] + outp_ref[1, :n_nodes, :]
    den = denp_ref[0, :n_nodes] + denp_ref[1, :n_nodes]
    c0 = num / (den + jnp.float32(1e-16))[:, None] + b0_ref[...][None, :]
    h1 = jax.nn.relu(c0 + hl0_ref[...])
    hs1 = jnp.dot(h1, w1_ref[...], preferred_element_type=jnp.float32)
    hs1_ref[...] = hs1
    as1_ref[...] = jnp.dot(hs1, atts_ref[...], preferred_element_type=jnp.float32)
    ad1_ref[...] = jnp.dot(hs1, attd_ref[...], preferred_element_type=jnp.float32)
    hl1_ref[...] = jnp.dot(h1, wlin1_ref[...], preferred_element_type=jnp.float32)


def _tc3_body(outp_ref, denp_ref, b1_ref, hl1_ref, lnw_ref, lnb_ref, wp_ref,
              out_ref, *, n_nodes):
    num = outp_ref[0, :n_nodes, :] + outp_ref[1, :n_nodes, :]
    den = denp_ref[0, :n_nodes] + denp_ref[1, :n_nodes]
    c1 = num / (den + jnp.float32(1e-16))[:, None] + b1_ref[...][None, :]
    h = c1 + hl1_ref[...]
    mu = jnp.mean(h)
    var = jnp.mean((h - mu) ** 2)
    h = (h - mu) / jnp.sqrt(var + jnp.float32(1e-5))
    h = h * lnw_ref[...][None, :] + lnb_ref[...][None, :]
    out_ref[...] = jnp.dot(h, wp_ref[...], preferred_element_type=jnp.float32)


def kernel(x, edge_index, W_embed, W_src0, W_dst0, att_src0, att_dst0, b0,
           W_lin0, W1, att_src1, att_dst1, b1, W_lin1, ln_w, ln_b, W_proj):
    n_nodes = x.shape[0]
    e_real = edge_index.shape[1]
    c0 = W_src0.shape[1]
    c1 = W1.shape[1]
    d_out = W_proj.shape[1]

    cpt = -(-e_real // (NW * K))
    e_pad = NW * cpt * K
    pad = e_pad - e_real
    src = edge_index[0]
    dst = edge_index[1]
    if pad:
        src = jnp.concatenate([src, jnp.arange(pad, dtype=jnp.int32) % n_nodes])
        dst = jnp.concatenate([dst, jnp.zeros((pad,), jnp.int32)])
    src3 = src.reshape(NW, cpt, K)
    dst3 = dst.reshape(NW, cpt, K)

    n_pad = NS * K * (-(-n_nodes // (NS * K)))

    hs0, as0, ad0, hl0 = pl.pallas_call(
        _tc1_body,
        out_shape=(
            jax.ShapeDtypeStruct((n_nodes, c0), jnp.float32),
            jax.ShapeDtypeStruct((n_nodes,), jnp.float32),
            jax.ShapeDtypeStruct((n_nodes,), jnp.float32),
            jax.ShapeDtypeStruct((n_nodes, c0), jnp.float32),
        ),
    )(x, W_embed, W_src0, W_dst0, W_lin0, att_src0, att_dst0)

    outp0, denp0 = _sc_gat_pass(hs0, as0, ad0, src3, dst3, n_pad, e_real)

    hs1, as1, ad1, hl1 = pl.pallas_call(
        functools.partial(_tc2_body, n_nodes=n_nodes),
        out_shape=(
            jax.ShapeDtypeStruct((n_nodes, c1), jnp.float32),
            jax.ShapeDtypeStruct((n_nodes,), jnp.float32),
            jax.ShapeDtypeStruct((n_nodes,), jnp.float32),
            jax.ShapeDtypeStruct((n_nodes, c1), jnp.float32),
        ),
    )(outp0, denp0, b0, hl0, W1, att_src1, att_dst1, W_lin1)

    outp1, denp1 = _sc_gat_pass(hs1, as1, ad1, src3, dst3, n_pad, e_real)

    out = pl.pallas_call(
        functools.partial(_tc3_body, n_nodes=n_nodes),
        out_shape=jax.ShapeDtypeStruct((n_nodes, d_out), jnp.float32),
    )(outp1, denp1, b1, hl1, ln_w, ln_b, W_proj)
    return out


# speedup vs baseline: 54.5227x; 1.7004x over previous
import functools

import jax
import jax.numpy as jnp
from jax import lax
from jax.experimental import pallas as pl
from jax.experimental.pallas import tpu as pltpu
from jax.experimental.pallas import tpu_sc as plsc

NC = 2
NS = 16
NW = NC * NS
K = 128
NSLOT = 6


def _sc_gat_pass(hs, asrc, adst, edges3, n_pad, e_real):
    n_nodes, feat = hs.shape
    cpt = edges3.shape[1]
    assert cpt % 2 == 0 and cpt >= 4
    ept = cpt * K
    rows_per_tile = n_pad // NS
    blocks_per_tile = rows_per_tile // K

    mesh = plsc.VectorSubcoreMesh(core_axis_name="c", subcore_axis_name="s")

    @functools.partial(
        pl.kernel,
        out_type=(
            jax.ShapeDtypeStruct((NC, n_pad, feat), jnp.float32),
            jax.ShapeDtypeStruct((NC, n_pad), jnp.float32),
        ),
        mesh=mesh,
        compiler_params=pltpu.CompilerParams(
            needs_layout_passes=False, use_tc_tiling_on_sc=False),
        scratch_types=dict(
            idx_r=pltpu.VMEM((NSLOT, 2, K), jnp.int32),
            ea_c=pltpu.VMEM((K,), jnp.float32),
            asg_v=pltpu.VMEM((K,), jnp.float32),
            adg_v=pltpu.VMEM((K,), jnp.float32),
            rows_v=pltpu.VMEM((2, K, feat), jnp.float32),
            zden_v=pltpu.VMEM((rows_per_tile,), jnp.float32),
            as_sh=pltpu.VMEM_SHARED((n_nodes,), jnp.float32),
            ad_sh=pltpu.VMEM_SHARED((n_nodes,), jnp.float32),
            out_acc=pltpu.VMEM_SHARED((n_pad, feat), jnp.float32),
            den_acc=pltpu.VMEM_SHARED((n_pad,), jnp.float32),
            g0_sem=pltpu.SemaphoreType.DMA,
            g1_sem=pltpu.SemaphoreType.DMA,
            sc0_sem=pltpu.SemaphoreType.DMA,
            sc1_sem=pltpu.SemaphoreType.DMA,
            i_sem=pltpu.SemaphoreType.DMA,
        ),
    )
    def gat_kernel(hs_hbm, as_hbm, ad_hbm, edges_hbm, outp_hbm, denp_hbm,
                   idx_r, ea_c, asg_v, adg_v, rows_v, zden_v,
                   as_sh, ad_sh, out_acc, den_acc,
                   g0_sem, g1_sem, sc0_sem, sc1_sem, i_sem):
        cidx = lax.axis_index("c")
        sidx = lax.axis_index("s")
        wid = cidx * NS + sidx
        base_gid = wid * ept

        @pl.when(sidx == 0)
        def _():
            pltpu.sync_copy(as_hbm, as_sh)
            pltpu.sync_copy(ad_hbm, ad_sh)

        pltpu.sync_copy(edges_hbm.at[wid, 0], idx_r.at[0])
        pltpu.sync_copy(edges_hbm.at[wid, 1], idx_r.at[1])
        pltpu.async_copy(edges_hbm.at[wid, 2], idx_r.at[2], i_sem)
        pltpu.async_copy(edges_hbm.at[wid, 3], idx_r.at[3], i_sem)

        zero16 = jnp.zeros((16,), jnp.float32)

        def zrow(r, _):
            for b in range(2):
                for v in range(feat // 16):
                    rows_v[b, r, pl.ds(v * 16, 16)] = zero16
            return 0
        lax.fori_loop(0, K, zrow, 0)

        def zden(i, _):
            zden_v[pl.ds(i * 16, 16)] = zero16
            return 0
        lax.fori_loop(0, rows_per_tile // 16, zden, 0)

        for b in range(blocks_per_tile):
            pltpu.sync_copy(rows_v.at[0],
                            out_acc.at[pl.ds(sidx * rows_per_tile + b * K, K)])
        pltpu.sync_copy(zden_v, den_acc.at[pl.ds(sidx * rows_per_tile, rows_per_tile)])
        plsc.subcore_barrier()

        iota16 = lax.iota(jnp.int32, 16)

        def compute_ea(j, slot):
            pltpu.sync_copy(as_sh.at[idx_r.at[slot, 0]], asg_v)
            pltpu.sync_copy(ad_sh.at[idx_r.at[slot, 1]], adg_v)
            for g in range(K // 16):
                a = asg_v[pl.ds(g * 16, 16)] + adg_v[pl.ds(g * 16, 16)]
                a = jnp.where(a >= 0.0, a, a * jnp.float32(0.2))
                ea = jnp.exp(a)
                gid = base_gid + j * K + g * 16 + iota16
                ea = jnp.where(gid < e_real, ea, jnp.float32(0.0))
                ea_c[pl.ds(g * 16, 16)] = ea

        def scale_rows(b):
            def grp(g, _):
                eag = ea_c[pl.ds(g * 16, 16)]
                for e in range(16):
                    sp = jnp.take_along_axis(
                        eag, jnp.full((16,), e, jnp.int32), axis=0)
                    r = g * 16 + e
                    for v in range(feat // 16):
                        rows_v[b, r, pl.ds(v * 16, 16)] = (
                            rows_v[b, r, pl.ds(v * 16, 16)] * sp)
                return 0
            lax.fori_loop(0, K // 16, grp, 0)

        pltpu.async_copy(hs_hbm.at[idx_r.at[0, 0]], rows_v.at[0], g0_sem)
        pltpu.async_copy(rows_v.at[1], out_acc.at[idx_r.at[0, 1]], sc1_sem,
                         add=True)

        def chunk2(jo, _):
            j0 = 2 * jo
            j1 = j0 + 1
            s0 = lax.rem(j0, NSLOT)
            s1 = lax.rem(j1, NSLOT)
            n0 = lax.rem(j0 + 2, NSLOT)
            n1 = lax.rem(j1 + 2, NSLOT)

            compute_ea(j0, s0)
            pltpu.make_async_copy(hs_hbm.at[idx_r.at[s0, 0]], rows_v.at[0],
                                  g0_sem).wait()
            pltpu.make_async_copy(rows_v.at[1], out_acc.at[idx_r.at[s0, 1]],
                                  sc1_sem).wait()
            g1 = pltpu.async_copy(hs_hbm.at[idx_r.at[s1, 0]], rows_v.at[1],
                                  g1_sem)
            scale_rows(0)
            s0scat = pltpu.async_copy(rows_v.at[0], out_acc.at[idx_r.at[s0, 1]],
                                      sc0_sem, add=True)
            pltpu.sync_copy(ea_c, den_acc.at[idx_r.at[s0, 1]], add=True)

            compute_ea(j1, s1)
            g1.wait()
            s0scat.wait()

            @pl.when(j0 + 2 < cpt)
            def _():
                pltpu.make_async_copy(edges_hbm.at[wid, j0 + 2], idx_r.at[n0],
                                      i_sem).wait()
                pltpu.make_async_copy(edges_hbm.at[wid, j1 + 2], idx_r.at[n1],
                                      i_sem).wait()
                pltpu.async_copy(hs_hbm.at[idx_r.at[n0, 0]], rows_v.at[0],
                                 g0_sem)

                @pl.when(j0 + 4 < cpt)
                def _():
                    pltpu.async_copy(edges_hbm.at[wid, j0 + 4],
                                     idx_r.at[lax.rem(j0 + 4, NSLOT)], i_sem)
                    pltpu.async_copy(edges_hbm.at[wid, j1 + 4],
                                     idx_r.at[lax.rem(j1 + 4, NSLOT)], i_sem)

            scale_rows(1)
            pltpu.async_copy(rows_v.at[1], out_acc.at[idx_r.at[s1, 1]],
                             sc1_sem, add=True)
            pltpu.sync_copy(ea_c, den_acc.at[idx_r.at[s1, 1]], add=True)
            return 0

        lax.fori_loop(0, cpt // 2, chunk2, 0)
        pltpu.make_async_copy(rows_v.at[1],
                              out_acc.at[idx_r.at[lax.rem(cpt - 1, NSLOT), 1]],
                              sc1_sem).wait()
        plsc.subcore_barrier()

        pltpu.sync_copy(out_acc.at[pl.ds(sidx * rows_per_tile, rows_per_tile)],
                        outp_hbm.at[cidx, pl.ds(sidx * rows_per_tile, rows_per_tile)])
        pltpu.sync_copy(den_acc.at[pl.ds(sidx * rows_per_tile, rows_per_tile)],
                        denp_hbm.at[cidx, pl.ds(sidx * rows_per_tile, rows_per_tile)])

    return gat_kernel(hs, asrc, adst, edges3)


def _tc1_body(x_ref, we_ref, ws_ref, wd_ref, wl_ref, atts_ref, attd_ref,
              hs_ref, as_ref, ad_ref, hl_ref):
    h = jnp.dot(x_ref[...], we_ref[...], preferred_element_type=jnp.float32)
    hs = jnp.dot(h, ws_ref[...], preferred_element_type=jnp.float32)
    hs_ref[...] = hs
    as_ref[...] = jnp.dot(hs, atts_ref[...][:, None],
                          preferred_element_type=jnp.float32)
    vd = jnp.dot(wd_ref[...], attd_ref[...][:, None],
                 preferred_element_type=jnp.float32)
    ad_ref[...] = jnp.dot(h, vd, preferred_element_type=jnp.float32)
    hl_ref[...] = jnp.dot(h, wl_ref[...], preferred_element_type=jnp.float32)


def _tc2_body(outp_ref, denp_ref, b0_ref, hl0_ref, w1_ref, atts_ref, attd_ref,
              wlin1_ref, hs1_ref, as1_ref, ad1_ref, hl1_ref, *, n_nodes):
    num = outp_ref[0, :n_nodes, :] + outp_ref[1, :n_nodes, :]
    den = denp_ref[0, :n_nodes] + denp_ref[1, :n_nodes]
    c0 = num / (den + jnp.float32(1e-16))[:, None] + b0_ref[...][None, :]
    h1 = jax.nn.relu(c0 + hl0_ref[...])
    hs1 = jnp.dot(h1, w1_ref[...], preferred_element_type=jnp.float32)
    hs1_ref[...] = hs1
    as1_ref[...] = jnp.dot(hs1, atts_ref[...][:, None],
                           preferred_element_type=jnp.float32)
    ad1_ref[...] = jnp.dot(hs1, attd_ref[...][:, None],
                           preferred_element_type=jnp.float32)
    hl1_ref[...] = jnp.dot(h1, wlin1_ref[...], preferred_element_type=jnp.float32)


def _tc3_body(outp_ref, denp_ref, b1_ref, hl1_ref, lnw_ref, lnb_ref, wp_ref,
              out_ref, *, n_nodes):
    num = outp_ref[0, :n_nodes, :] + outp_ref[1, :n_nodes, :]
    den = denp_ref[0, :n_nodes] + denp_ref[1, :n_nodes]
    c1 = num / (den + jnp.float32(1e-16))[:, None] + b1_ref[...][None, :]
    h = c1 + hl1_ref[...]
    mu = jnp.mean(h)
    var = jnp.mean((h - mu) ** 2)
    h = (h - mu) / jnp.sqrt(var + jnp.float32(1e-5))
    h = h * lnw_ref[...][None, :] + lnb_ref[...][None, :]
    out_ref[...] = jnp.dot(h, wp_ref[...], preferred_element_type=jnp.float32)


def kernel(x, edge_index, W_embed, W_src0, W_dst0, att_src0, att_dst0, b0,
           W_lin0, W1, att_src1, att_dst1, b1, W_lin1, ln_w, ln_b, W_proj):
    n_nodes = x.shape[0]
    e_real = edge_index.shape[1]
    c0 = W_src0.shape[1]
    c1 = W1.shape[1]
    d_out = W_proj.shape[1]

    cpt = -(-e_real // (NW * K))
    cpt += cpt % 2
    e_pad = NW * cpt * K
    pad = e_pad - e_real
    src = edge_index[0]
    dst = edge_index[1]
    if pad:
        src = jnp.concatenate([src, jnp.arange(pad, dtype=jnp.int32) % n_nodes])
        dst = jnp.concatenate([dst, jnp.zeros((pad,), jnp.int32)])
    edges3 = jnp.stack([src.reshape(NW, cpt, K), dst.reshape(NW, cpt, K)],
                       axis=2)

    n_pad = NS * K * (-(-n_nodes // (NS * K)))

    hs0, as0, ad0, hl0 = pl.pallas_call(
        _tc1_body,
        out_shape=(
            jax.ShapeDtypeStruct((n_nodes, c0), jnp.float32),
            jax.ShapeDtypeStruct((n_nodes, 1), jnp.float32),
            jax.ShapeDtypeStruct((n_nodes, 1), jnp.float32),
            jax.ShapeDtypeStruct((n_nodes, c0), jnp.float32),
        ),
    )(x, W_embed, W_src0, W_dst0, W_lin0, att_src0, att_dst0)
    as0 = as0[:, 0]
    ad0 = ad0[:, 0]

    outp0, denp0 = _sc_gat_pass(hs0, as0, ad0, edges3, n_pad, e_real)

    hs1, as1, ad1, hl1 = pl.pallas_call(
        functools.partial(_tc2_body, n_nodes=n_nodes),
        out_shape=(
            jax.ShapeDtypeStruct((n_nodes, c1), jnp.float32),
            jax.ShapeDtypeStruct((n_nodes, 1), jnp.float32),
            jax.ShapeDtypeStruct((n_nodes, 1), jnp.float32),
            jax.ShapeDtypeStruct((n_nodes, c1), jnp.float32),
        ),
    )(outp0, denp0, b0, hl0, W1, att_src1, att_dst1, W_lin1)
    as1 = as1[:, 0]
    ad1 = ad1[:, 0]

    outp1, denp1 = _sc_gat_pass(hs1, as1, ad1, edges3, n_pad, e_real)

    out = pl.pallas_call(
        functools.partial(_tc3_body, n_nodes=n_nodes),
        out_shape=jax.ShapeDtypeStruct((n_nodes, d_out), jnp.float32),
    )(outp1, denp1, b1, hl1, ln_w, ln_b, W_proj)
    return out
